# baseline (speedup 1.0000x reference)
import functools

import numpy as np
import jax
import jax.numpy as jnp
from jax import lax
from jax.experimental import pallas as pl
from jax.experimental.pallas import tpu as pltpu
from jax.experimental.pallas import tpu_sc as plsc

INPUT_DIM = 3
NUM_LEVELS = 16
LEVEL_DIM = 2
BASE_RES = 16
LOG2_HASHMAP_SIZE = 19
N_POINTS = 524288

P1 = np.int32(np.uint32(2654435761).astype(np.int64) - (1 << 32))
P2 = np.int32(805459861)
HASH_MASK = (1 << LOG2_HASHMAP_SIZE) - 1


def _level_table():
    max_params = 2 ** LOG2_HASHMAP_SIZE
    levels = []
    offset = 0
    for lvl in range(NUM_LEVELS):
        table_res = int(np.ceil(BASE_RES * 2.0 ** lvl)) + 1
        params = min(max_params, table_res ** INPUT_DIM)
        params = int(np.ceil(params / 8) * 8)
        scale = float(np.exp2(lvl) * BASE_RES - 1.0)
        res = int(np.ceil(scale)) + 1
        use_hash = (res ** INPUT_DIM) > params
        levels.append((scale, offset, params, use_hash, res))
        offset += params
    return levels, offset


LEVELS, TOTAL_ROWS = _level_table()

NW = 32
PW = N_POINTS // NW
C = 1024
NG = C // 16
NCH = PW // C
OUT_DIM = NUM_LEVELS * LEVEL_DIM


def _corner_indices(ux, uy, uz, level):
    scale, off, hsize, use_hash, res = LEVELS[level]
    if use_hash:
        hx = (ux, ux + 1)
        hy0 = uy * P1
        hz0 = uz * P2
        hy = (hy0, hy0 + P1)
        hz = (hz0, hz0 + P2)
        hyz = [[hy[by] ^ hz[bz] for bz in range(2)] for by in range(2)]
        out = []
        for c in range(8):
            b0, b1, b2 = c & 1, (c >> 1) & 1, (c >> 2) & 1
            out.append(((hx[b0] ^ hyz[b1][b2]) & HASH_MASK) + off)
        return out
    e1 = np.int32(res)
    e2 = np.int32(res * res)
    cx = (ux, ux + 1)
    ty0 = uy * e1
    tz0 = uz * e2
    ty = (ty0, ty0 + e1)
    tz = (tz0, tz0 + e2)
    tyz = [[ty[by] + tz[bz] for bz in range(2)] for by in range(2)]
    out = []
    for c in range(8):
        b0, b1, b2 = c & 1, (c >> 1) & 1, (c >> 2) & 1
        idx = cx[b0] + tyz[b1][b2]
        idx = jnp.where(idx >= hsize, idx - hsize, idx)
        out.append(idx + off)
    return out


def _make_grid_kernel():
    mesh = plsc.VectorSubcoreMesh(core_axis_name="c", subcore_axis_name="s")

    @functools.partial(
        pl.kernel,
        mesh=mesh,
        compiler_params=pltpu.CompilerParams(needs_layout_passes=False),
        out_type=jax.ShapeDtypeStruct((N_POINTS * OUT_DIM,), jnp.float32),
        scratch_types=[
            pltpu.VMEM((C,), jnp.float32),
            pltpu.VMEM((C,), jnp.float32),
            pltpu.VMEM((C,), jnp.float32),
            pltpu.VMEM((16 * C,), jnp.int32),
            pltpu.VMEM((8 * C,), jnp.float32),
            pltpu.VMEM((16 * C,), jnp.float32),
            pltpu.VMEM((C * OUT_DIM,), jnp.float32),
            pltpu.SemaphoreType.DMA,
        ],
    )
    def grid_kernel(xs_h, ys_h, zs_h, tab_h, out_h,
                    xs_v, ys_v, zs_v, idx_v, w_v, rows_v, ob_v, sem):
        wid = lax.axis_index("c") * 16 + lax.axis_index("s")
        iota = lax.iota(jnp.int32, 16)
        orow = iota * OUT_DIM

        def chunk_body(ch, carry):
            base = wid * PW + ch * C
            pltpu.sync_copy(xs_h.at[pl.ds(base, C)], xs_v)
            pltpu.sync_copy(ys_h.at[pl.ds(base, C)], ys_v)
            pltpu.sync_copy(zs_h.at[pl.ds(base, C)], zs_v)

            for level in range(NUM_LEVELS):
                a = np.float32(LEVELS[level][0])
                half = np.float32(0.5)
                one = np.float32(1.0)

                def grp_body(g, carry2, level=level, a=a):
                    p0 = g * 16
                    xv = xs_v[pl.ds(p0, 16)]
                    yv = ys_v[pl.ds(p0, 16)]
                    zv = zs_v[pl.ds(p0, 16)]
                    px = ((xv + one) * half) * a + half
                    py = ((yv + one) * half) * a + half
                    pz = ((zv + one) * half) * a + half
                    ux = px.astype(jnp.int32)
                    uy = py.astype(jnp.int32)
                    uz = pz.astype(jnp.int32)
                    fx = px - ux.astype(jnp.float32)
                    fy = py - uy.astype(jnp.float32)
                    fz = pz - uz.astype(jnp.float32)
                    idxs = _corner_indices(ux, uy, uz, level)
                    gx = (1.0 - fx, fx)
                    gy = (1.0 - fy, fy)
                    gz = (1.0 - fz, fz)
                    wxy = [[gx[b0] * gy[b1] for b1 in range(2)]
                           for b0 in range(2)]
                    for c in range(8):
                        b0, b1, b2 = c & 1, (c >> 1) & 1, (c >> 2) & 1
                        w_c = wxy[b0][b1] * gz[b2]
                        e0 = idxs[c] * 2
                        idx_v[pl.ds(c * C + p0, 16)] = e0
                        idx_v[pl.ds(8 * C + c * C + p0, 16)] = e0 + 1
                        w_v[pl.ds(c * C + p0, 16)] = w_c
                    return carry2

                lax.fori_loop(0, NG, grp_body, 0, unroll=False)

                pltpu.async_copy(tab_h.at[idx_v], rows_v, sem).wait()

                obase = 2 * level

                def acc_body(g, carry2, obase=obase):
                    p0 = g * 16
                    acc0 = jnp.zeros((16,), jnp.float32)
                    acc1 = jnp.zeros((16,), jnp.float32)
                    for c in range(8):
                        rbase = c * C + p0
                        wv = w_v[pl.ds(rbase, 16)]
                        f0 = rows_v[pl.ds(rbase, 16)]
                        f1 = rows_v[pl.ds(8 * C + rbase, 16)]
                        acc0 = acc0 + wv * f0
                        acc1 = acc1 + wv * f1
                    opos = p0 * OUT_DIM + obase + orow
                    plsc.store_scatter(ob_v, [opos], acc0)
                    plsc.store_scatter(ob_v, [opos + 1], acc1)
                    return carry2

                lax.fori_loop(0, NG, acc_body, 0, unroll=False)

            pltpu.sync_copy(ob_v, out_h.at[pl.ds(base * OUT_DIM, C * OUT_DIM)])
            return carry

        lax.fori_loop(0, NCH, chunk_body, 0, unroll=False)

    return grid_kernel


_GRID_KERNEL_CACHE = []


def kernel(inputs, embeddings):
    if not _GRID_KERNEL_CACHE:
        _GRID_KERNEL_CACHE.append(_make_grid_kernel())
    xyz = inputs.T
    flat = _GRID_KERNEL_CACHE[0](xyz[0], xyz[1], xyz[2], embeddings.reshape(-1))
    return flat.reshape(N_POINTS, OUT_DIM)


# speedup vs baseline: 3.2101x; 3.2101x over previous
import functools

import numpy as np
import jax
import jax.numpy as jnp
from jax import lax
from jax.experimental import pallas as pl
from jax.experimental.pallas import tpu as pltpu
from jax.experimental.pallas import tpu_sc as plsc

INPUT_DIM = 3
NUM_LEVELS = 16
LEVEL_DIM = 2
BASE_RES = 16
LOG2_HASHMAP_SIZE = 19
N_POINTS = 524288

P1 = np.int32(np.uint32(2654435761).astype(np.int64) - (1 << 32))
P2 = np.int32(805459861)
HASH_MASK = (1 << LOG2_HASHMAP_SIZE) - 1


def _level_table():
    max_params = 2 ** LOG2_HASHMAP_SIZE
    levels = []
    offset = 0
    for lvl in range(NUM_LEVELS):
        table_res = int(np.ceil(BASE_RES * 2.0 ** lvl)) + 1
        params = min(max_params, table_res ** INPUT_DIM)
        params = int(np.ceil(params / 8) * 8)
        scale = float(np.exp2(lvl) * BASE_RES - 1.0)
        res = int(np.ceil(scale)) + 1
        use_hash = (res ** INPUT_DIM) > params
        levels.append((scale, offset, params, use_hash, res))
        offset += params
    return levels, offset


LEVELS, TOTAL_ROWS = _level_table()

NW = 32
PW = N_POINTS // NW
C = 1024
NG = C // 16
NCH = PW // C
OUT_DIM = NUM_LEVELS * LEVEL_DIM


def _corner_indices(ux, uy, uz, level):
    scale, off, hsize, use_hash, res = LEVELS[level]
    if use_hash:
        hx = (ux, ux + 1)
        hy0 = uy * P1
        hz0 = uz * P2
        hy = (hy0, hy0 + P1)
        hz = (hz0, hz0 + P2)
        hyz = [[hy[by] ^ hz[bz] for bz in range(2)] for by in range(2)]
        out = []
        for c in range(8):
            b0, b1, b2 = c & 1, (c >> 1) & 1, (c >> 2) & 1
            out.append(((hx[b0] ^ hyz[b1][b2]) & HASH_MASK) + off)
        return out
    e1 = np.int32(res)
    e2 = np.int32(res * res)
    cx = (ux, ux + 1)
    ty0 = uy * e1
    tz0 = uz * e2
    ty = (ty0, ty0 + e1)
    tz = (tz0, tz0 + e2)
    tyz = [[ty[by] + tz[bz] for bz in range(2)] for by in range(2)]
    out = []
    for c in range(8):
        b0, b1, b2 = c & 1, (c >> 1) & 1, (c >> 2) & 1
        idx = cx[b0] + tyz[b1][b2]
        idx = jnp.where(idx >= hsize, idx - hsize, idx)
        out.append(idx + off)
    return out


def _make_grid_kernel():
    mesh = plsc.VectorSubcoreMesh(core_axis_name="c", subcore_axis_name="s")

    @functools.partial(
        pl.kernel,
        mesh=mesh,
        compiler_params=pltpu.CompilerParams(needs_layout_passes=False),
        out_type=jax.ShapeDtypeStruct((N_POINTS * OUT_DIM,), jnp.float32),
        scratch_types=[
            pltpu.VMEM((C,), jnp.float32),
            pltpu.VMEM((C,), jnp.float32),
            pltpu.VMEM((C,), jnp.float32),
            pltpu.VMEM((8 * C,), jnp.int32),
            pltpu.VMEM((8 * C,), jnp.float32),
            pltpu.VMEM((8 * C,), jnp.int32),
            pltpu.VMEM((C * OUT_DIM,), jnp.float32),
            pltpu.SemaphoreType.DMA,
        ],
    )
    def grid_kernel(xs_h, ys_h, zs_h, tab_h, out_h,
                    xs_v, ys_v, zs_v, idx_v, w_v, rows_v, ob_v, sem):
        wid = lax.axis_index("c") * 16 + lax.axis_index("s")
        iota = lax.iota(jnp.int32, 16)
        orow = iota * OUT_DIM

        def chunk_body(ch, carry):
            base = wid * PW + ch * C
            pltpu.sync_copy(xs_h.at[pl.ds(base, C)], xs_v)
            pltpu.sync_copy(ys_h.at[pl.ds(base, C)], ys_v)
            pltpu.sync_copy(zs_h.at[pl.ds(base, C)], zs_v)

            for level in range(NUM_LEVELS):
                a = np.float32(LEVELS[level][0])
                half = np.float32(0.5)
                one = np.float32(1.0)

                def grp_body(g, carry2, level=level, a=a):
                    p0 = g * 16
                    xv = xs_v[pl.ds(p0, 16)]
                    yv = ys_v[pl.ds(p0, 16)]
                    zv = zs_v[pl.ds(p0, 16)]
                    px = ((xv + one) * half) * a + half
                    py = ((yv + one) * half) * a + half
                    pz = ((zv + one) * half) * a + half
                    ux = px.astype(jnp.int32)
                    uy = py.astype(jnp.int32)
                    uz = pz.astype(jnp.int32)
                    fx = px - ux.astype(jnp.float32)
                    fy = py - uy.astype(jnp.float32)
                    fz = pz - uz.astype(jnp.float32)
                    idxs = _corner_indices(ux, uy, uz, level)
                    gx = (1.0 - fx, fx)
                    gy = (1.0 - fy, fy)
                    gz = (1.0 - fz, fz)
                    wxy = [[gx[b0] * gy[b1] for b1 in range(2)]
                           for b0 in range(2)]
                    for c in range(8):
                        b0, b1, b2 = c & 1, (c >> 1) & 1, (c >> 2) & 1
                        w_c = wxy[b0][b1] * gz[b2]
                        idx_v[pl.ds(c * C + p0, 16)] = idxs[c]
                        w_v[pl.ds(c * C + p0, 16)] = w_c
                    return carry2

                lax.fori_loop(0, NG, grp_body, 0, unroll=False)

                pltpu.async_copy(tab_h.at[idx_v], rows_v, sem).wait()

                obase = 2 * level

                def acc_body(g, carry2, obase=obase):
                    p0 = g * 16
                    acc0 = jnp.zeros((16,), jnp.float32)
                    acc1 = jnp.zeros((16,), jnp.float32)
                    for c in range(8):
                        rbase = c * C + p0
                        wv = w_v[pl.ds(rbase, 16)]
                        gv = rows_v[pl.ds(rbase, 16)]
                        f0, f1 = plsc.unpack(
                            plsc.bitcast(gv, jnp.bfloat16),
                            format=plsc.PackFormat.INTERLEAVED)
                        acc0 = acc0 + wv * f0
                        acc1 = acc1 + wv * f1
                    opos = p0 * OUT_DIM + obase + orow
                    plsc.store_scatter(ob_v, [opos], acc0)
                    plsc.store_scatter(ob_v, [opos + 1], acc1)
                    return carry2

                lax.fori_loop(0, NG, acc_body, 0, unroll=False)

            pltpu.sync_copy(ob_v, out_h.at[pl.ds(base * OUT_DIM, C * OUT_DIM)])
            return carry

        lax.fori_loop(0, NCH, chunk_body, 0, unroll=False)

    return grid_kernel


_GRID_KERNEL_CACHE = []


def kernel(inputs, embeddings):
    if not _GRID_KERNEL_CACHE:
        _GRID_KERNEL_CACHE.append(_make_grid_kernel())
    xyz = inputs.T
    packed = lax.bitcast_convert_type(
        embeddings.astype(jnp.bfloat16), jnp.int32)
    flat = _GRID_KERNEL_CACHE[0](xyz[0], xyz[1], xyz[2], packed)
    return flat.reshape(N_POINTS, OUT_DIM)


# speedup vs baseline: 3.4394x; 1.0714x over previous
import functools

import numpy as np
import jax
import jax.numpy as jnp
from jax import lax
from jax.experimental import pallas as pl
from jax.experimental.pallas import tpu as pltpu
from jax.experimental.pallas import tpu_sc as plsc

INPUT_DIM = 3
NUM_LEVELS = 16
LEVEL_DIM = 2
BASE_RES = 16
LOG2_HASHMAP_SIZE = 19
N_POINTS = 524288

P1 = np.int32(np.uint32(2654435761).astype(np.int64) - (1 << 32))
P2 = np.int32(805459861)
HASH_MASK = (1 << LOG2_HASHMAP_SIZE) - 1


def _level_table():
    max_params = 2 ** LOG2_HASHMAP_SIZE
    levels = []
    offset = 0
    for lvl in range(NUM_LEVELS):
        table_res = int(np.ceil(BASE_RES * 2.0 ** lvl)) + 1
        params = min(max_params, table_res ** INPUT_DIM)
        params = int(np.ceil(params / 8) * 8)
        scale = float(np.exp2(lvl) * BASE_RES - 1.0)
        res = int(np.ceil(scale)) + 1
        use_hash = (res ** INPUT_DIM) > params
        levels.append((scale, offset, params, use_hash, res))
        offset += params
    return levels, offset


LEVELS, TOTAL_ROWS = _level_table()

NW = 32
PW = N_POINTS // NW
C = 1024
NG = C // 16
NCH = PW // C
OUT_DIM = NUM_LEVELS * LEVEL_DIM


def _corner_indices(ux, uy, uz, level):
    scale, off, hsize, use_hash, res = LEVELS[level]
    if use_hash:
        hx = (ux, ux + 1)
        hy0 = uy * P1
        hz0 = uz * P2
        hy = (hy0, hy0 + P1)
        hz = (hz0, hz0 + P2)
        hyz = [[hy[by] ^ hz[bz] for bz in range(2)] for by in range(2)]
        out = []
        for c in range(8):
            b0, b1, b2 = c & 1, (c >> 1) & 1, (c >> 2) & 1
            out.append(((hx[b0] ^ hyz[b1][b2]) & HASH_MASK) + off)
        return out
    e1 = np.int32(res)
    e2 = np.int32(res * res)
    cx = (ux, ux + 1)
    ty0 = uy * e1
    tz0 = uz * e2
    ty = (ty0, ty0 + e1)
    tz = (tz0, tz0 + e2)
    tyz = [[ty[by] + tz[bz] for bz in range(2)] for by in range(2)]
    out = []
    for c in range(8):
        b0, b1, b2 = c & 1, (c >> 1) & 1, (c >> 2) & 1
        idx = cx[b0] + tyz[b1][b2]
        idx = jnp.where(idx >= hsize, idx - hsize, idx)
        out.append(idx + off)
    return out


def _make_grid_kernel():
    mesh = plsc.VectorSubcoreMesh(core_axis_name="c", subcore_axis_name="s")

    @functools.partial(
        pl.kernel,
        mesh=mesh,
        compiler_params=pltpu.CompilerParams(needs_layout_passes=False),
        out_type=jax.ShapeDtypeStruct((N_POINTS * OUT_DIM,), jnp.float32),
        scratch_types=[
            pltpu.VMEM((C,), jnp.float32),
            pltpu.VMEM((C,), jnp.float32),
            pltpu.VMEM((C,), jnp.float32),
            pltpu.VMEM((8 * C,), jnp.int32),
            pltpu.VMEM((8 * C,), jnp.int32),
            pltpu.VMEM((8 * C,), jnp.float32),
            pltpu.VMEM((8 * C,), jnp.float32),
            pltpu.VMEM((8 * C,), jnp.int32),
            pltpu.VMEM((8 * C,), jnp.int32),
            pltpu.VMEM((C * OUT_DIM,), jnp.float32),
            pltpu.SemaphoreType.DMA,
            pltpu.SemaphoreType.DMA,
        ],
    )
    def grid_kernel(xs_h, ys_h, zs_h, tab_h, out_h,
                    xs_v, ys_v, zs_v, idx_a, idx_b, w_a, w_b,
                    rows_a, rows_b, ob_v, sem_a, sem_b):
        wid = lax.axis_index("c") * 16 + lax.axis_index("s")
        iota = lax.iota(jnp.int32, 16)
        orow = iota * OUT_DIM
        half = np.float32(0.5)
        one = np.float32(1.0)
        bufs = ((idx_a, w_a, rows_a, sem_a), (idx_b, w_b, rows_b, sem_b))

        def compute_pass(level, idx_v, w_v):
            a = np.float32(LEVELS[level][0])

            def grp_body(g, carry2):
                p0 = g * 16
                xv = xs_v[pl.ds(p0, 16)]
                yv = ys_v[pl.ds(p0, 16)]
                zv = zs_v[pl.ds(p0, 16)]
                px = ((xv + one) * half) * a + half
                py = ((yv + one) * half) * a + half
                pz = ((zv + one) * half) * a + half
                ux = px.astype(jnp.int32)
                uy = py.astype(jnp.int32)
                uz = pz.astype(jnp.int32)
                fx = px - ux.astype(jnp.float32)
                fy = py - uy.astype(jnp.float32)
                fz = pz - uz.astype(jnp.float32)
                idxs = _corner_indices(ux, uy, uz, level)
                gx = (1.0 - fx, fx)
                gy = (1.0 - fy, fy)
                gz = (1.0 - fz, fz)
                wxy = [[gx[b0] * gy[b1] for b1 in range(2)]
                       for b0 in range(2)]
                for c in range(8):
                    b0, b1, b2 = c & 1, (c >> 1) & 1, (c >> 2) & 1
                    w_c = wxy[b0][b1] * gz[b2]
                    idx_v[pl.ds(c * C + p0, 16)] = idxs[c]
                    w_v[pl.ds(c * C + p0, 16)] = w_c
                return carry2

            lax.fori_loop(0, NG, grp_body, 0, unroll=False)

        def acc_pass(level, rows_v, w_v):
            obase = 2 * level

            def acc_body(g, carry2):
                p0 = g * 16
                acc0 = jnp.zeros((16,), jnp.float32)
                acc1 = jnp.zeros((16,), jnp.float32)
                for c in range(8):
                    rbase = c * C + p0
                    wv = w_v[pl.ds(rbase, 16)]
                    gv = rows_v[pl.ds(rbase, 16)]
                    f0, f1 = plsc.unpack(
                        plsc.bitcast(gv, jnp.bfloat16),
                        format=plsc.PackFormat.INTERLEAVED)
                    acc0 = acc0 + wv * f0
                    acc1 = acc1 + wv * f1
                opos = p0 * OUT_DIM + obase + orow
                plsc.store_scatter(ob_v, [opos], acc0)
                plsc.store_scatter(ob_v, [opos + 1], acc1)
                return carry2

            lax.fori_loop(0, NG, acc_body, 0, unroll=False)

        def chunk_body(ch, carry):
            base = wid * PW + ch * C
            pltpu.sync_copy(xs_h.at[pl.ds(base, C)], xs_v)
            pltpu.sync_copy(ys_h.at[pl.ds(base, C)], ys_v)
            pltpu.sync_copy(zs_h.at[pl.ds(base, C)], zs_v)

            compute_pass(0, bufs[0][0], bufs[0][1])
            cp = pltpu.async_copy(tab_h.at[bufs[0][0]], bufs[0][2],
                                  bufs[0][3])
            for level in range(NUM_LEVELS):
                cur = bufs[level % 2]
                ncp = None
                if level + 1 < NUM_LEVELS:
                    nxt = bufs[(level + 1) % 2]
                    compute_pass(level + 1, nxt[0], nxt[1])
                    ncp = pltpu.async_copy(tab_h.at[nxt[0]], nxt[2], nxt[3])
                cp.wait()
                acc_pass(level, cur[2], cur[1])
                cp = ncp

            pltpu.sync_copy(ob_v, out_h.at[pl.ds(base * OUT_DIM, C * OUT_DIM)])
            return carry

        lax.fori_loop(0, NCH, chunk_body, 0, unroll=False)

    return grid_kernel


_GRID_KERNEL_CACHE = []


def kernel(inputs, embeddings):
    if not _GRID_KERNEL_CACHE:
        _GRID_KERNEL_CACHE.append(_make_grid_kernel())
    xyz = inputs.T
    packed = lax.bitcast_convert_type(
        embeddings.astype(jnp.bfloat16), jnp.int32)
    flat = _GRID_KERNEL_CACHE[0](xyz[0], xyz[1], xyz[2], packed)
    return flat.reshape(N_POINTS, OUT_DIM)


# speedup vs baseline: 3.4706x; 1.0091x over previous
import functools

import numpy as np
import jax
import jax.numpy as jnp
from jax import lax
from jax.experimental import pallas as pl
from jax.experimental.pallas import tpu as pltpu
from jax.experimental.pallas import tpu_sc as plsc

INPUT_DIM = 3
NUM_LEVELS = 16
LEVEL_DIM = 2
BASE_RES = 16
LOG2_HASHMAP_SIZE = 19
N_POINTS = 524288

P1 = np.int32(np.uint32(2654435761).astype(np.int64) - (1 << 32))
P2 = np.int32(805459861)
HASH_MASK = (1 << LOG2_HASHMAP_SIZE) - 1


def _level_table():
    max_params = 2 ** LOG2_HASHMAP_SIZE
    levels = []
    offset = 0
    for lvl in range(NUM_LEVELS):
        table_res = int(np.ceil(BASE_RES * 2.0 ** lvl)) + 1
        params = min(max_params, table_res ** INPUT_DIM)
        params = int(np.ceil(params / 8) * 8)
        scale = float(np.exp2(lvl) * BASE_RES - 1.0)
        res = int(np.ceil(scale)) + 1
        use_hash = (res ** INPUT_DIM) > params
        levels.append((scale, offset, params, use_hash, res))
        offset += params
    return levels, offset


LEVELS, TOTAL_ROWS = _level_table()

NW = 32
PW = N_POINTS // NW
C = 1024
NG = C // 16
NCH = PW // C
OUT_DIM = NUM_LEVELS * LEVEL_DIM


def _corner_indices(ux, uy, uz, level):
    scale, off, hsize, use_hash, res = LEVELS[level]
    if use_hash:
        hx = (ux, ux + 1)
        hy0 = uy * P1
        hz0 = uz * P2
        hy = (hy0, hy0 + P1)
        hz = (hz0, hz0 + P2)
        hyz = [[hy[by] ^ hz[bz] for bz in range(2)] for by in range(2)]
        out = []
        for c in range(8):
            b0, b1, b2 = c & 1, (c >> 1) & 1, (c >> 2) & 1
            out.append(((hx[b0] ^ hyz[b1][b2]) & HASH_MASK) + off)
        return out
    e1 = np.int32(res)
    e2 = np.int32(res * res)
    cx = (ux, ux + 1)
    ty0 = uy * e1
    tz0 = uz * e2
    ty = (ty0, ty0 + e1)
    tz = (tz0, tz0 + e2)
    tyz = [[ty[by] + tz[bz] for bz in range(2)] for by in range(2)]
    out = []
    for c in range(8):
        b0, b1, b2 = c & 1, (c >> 1) & 1, (c >> 2) & 1
        idx = cx[b0] + tyz[b1][b2]
        idx = jnp.where(idx >= hsize, idx - hsize, idx)
        out.append(idx + off)
    return out


def _make_grid_kernel():
    mesh = plsc.VectorSubcoreMesh(core_axis_name="c", subcore_axis_name="s")

    @functools.partial(
        pl.kernel,
        mesh=mesh,
        compiler_params=pltpu.CompilerParams(needs_layout_passes=False),
        out_type=jax.ShapeDtypeStruct((N_POINTS * OUT_DIM,), jnp.float32),
        scratch_types=[
            pltpu.VMEM((C,), jnp.float32),
            pltpu.VMEM((C,), jnp.float32),
            pltpu.VMEM((C,), jnp.float32),
            pltpu.VMEM((8 * C,), jnp.int32),
            pltpu.VMEM((8 * C,), jnp.int32),
            pltpu.VMEM((8 * C,), jnp.float32),
            pltpu.VMEM((8 * C,), jnp.float32),
            pltpu.VMEM((8 * C,), jnp.int32),
            pltpu.VMEM((8 * C,), jnp.int32),
            pltpu.VMEM((C * OUT_DIM,), jnp.float32),
            pltpu.SemaphoreType.DMA,
            pltpu.SemaphoreType.DMA,
        ],
    )
    def grid_kernel(xs_h, ys_h, zs_h, tab_h, out_h,
                    xs_v, ys_v, zs_v, idx_a, idx_b, w_a, w_b,
                    rows_a, rows_b, ob_v, sem_a, sem_b):
        wid = lax.axis_index("c") * 16 + lax.axis_index("s")
        iota = lax.iota(jnp.int32, 16)
        iota8 = iota * 8
        orow = iota * OUT_DIM
        half = np.float32(0.5)
        one = np.float32(1.0)
        bufs = ((idx_a, w_a, rows_a, sem_a), (idx_b, w_b, rows_b, sem_b))

        def compute_pass(level, idx_v, w_v):
            a = np.float32(LEVELS[level][0])

            def grp_body(g, carry2):
                p0 = g * 16
                xv = xs_v[pl.ds(p0, 16)]
                yv = ys_v[pl.ds(p0, 16)]
                zv = zs_v[pl.ds(p0, 16)]
                px = ((xv + one) * half) * a + half
                py = ((yv + one) * half) * a + half
                pz = ((zv + one) * half) * a + half
                ux = px.astype(jnp.int32)
                uy = py.astype(jnp.int32)
                uz = pz.astype(jnp.int32)
                fx = px - ux.astype(jnp.float32)
                fy = py - uy.astype(jnp.float32)
                fz = pz - uz.astype(jnp.float32)
                idxs = _corner_indices(ux, uy, uz, level)
                gx = (1.0 - fx, fx)
                gy = (1.0 - fy, fy)
                gz = (1.0 - fz, fz)
                wxy = [[gx[b0] * gy[b1] for b1 in range(2)]
                       for b0 in range(2)]
                gbase = p0 * 8 + iota8
                for c in range(8):
                    b0, b1, b2 = c & 1, (c >> 1) & 1, (c >> 2) & 1
                    w_c = wxy[b0][b1] * gz[b2]
                    plsc.store_scatter(idx_v, [gbase + c], idxs[c])
                    w_v[pl.ds(c * C + p0, 16)] = w_c
                return carry2

            lax.fori_loop(0, NG, grp_body, 0, unroll=False)

        def acc_pass(level, rows_v, w_v):
            obase = 2 * level

            def acc_body(g, carry2):
                p0 = g * 16
                acc0 = jnp.zeros((16,), jnp.float32)
                acc1 = jnp.zeros((16,), jnp.float32)
                gbase = p0 * 8 + iota8
                for c in range(8):
                    rbase = c * C + p0
                    wv = w_v[pl.ds(rbase, 16)]
                    gv = plsc.load_gather(rows_v, [gbase + c])
                    f0, f1 = plsc.unpack(
                        plsc.bitcast(gv, jnp.bfloat16),
                        format=plsc.PackFormat.INTERLEAVED)
                    acc0 = acc0 + wv * f0
                    acc1 = acc1 + wv * f1
                opos = p0 * OUT_DIM + obase + orow
                plsc.store_scatter(ob_v, [opos], acc0)
                plsc.store_scatter(ob_v, [opos + 1], acc1)
                return carry2

            lax.fori_loop(0, NG, acc_body, 0, unroll=False)

        def chunk_body(ch, carry):
            base = wid * PW + ch * C
            pltpu.sync_copy(xs_h.at[pl.ds(base, C)], xs_v)
            pltpu.sync_copy(ys_h.at[pl.ds(base, C)], ys_v)
            pltpu.sync_copy(zs_h.at[pl.ds(base, C)], zs_v)

            compute_pass(0, bufs[0][0], bufs[0][1])
            cp = pltpu.async_copy(tab_h.at[bufs[0][0]], bufs[0][2],
                                  bufs[0][3])
            for level in range(NUM_LEVELS):
                cur = bufs[level % 2]
                ncp = None
                if level + 1 < NUM_LEVELS:
                    nxt = bufs[(level + 1) % 2]
                    compute_pass(level + 1, nxt[0], nxt[1])
                    ncp = pltpu.async_copy(tab_h.at[nxt[0]], nxt[2], nxt[3])
                cp.wait()
                acc_pass(level, cur[2], cur[1])
                cp = ncp

            pltpu.sync_copy(ob_v, out_h.at[pl.ds(base * OUT_DIM, C * OUT_DIM)])
            return carry

        lax.fori_loop(0, NCH, chunk_body, 0, unroll=False)

    return grid_kernel


_GRID_KERNEL_CACHE = []


def kernel(inputs, embeddings):
    if not _GRID_KERNEL_CACHE:
        _GRID_KERNEL_CACHE.append(_make_grid_kernel())
    xyz = inputs.T
    packed = lax.bitcast_convert_type(
        embeddings.astype(jnp.bfloat16), jnp.int32)
    flat = _GRID_KERNEL_CACHE[0](xyz[0], xyz[1], xyz[2], packed)
    return flat.reshape(N_POINTS, OUT_DIM)


# speedup vs baseline: 6.1156x; 1.7621x over previous
import functools

import numpy as np
import jax
import jax.numpy as jnp
from jax import lax
from jax.experimental import pallas as pl
from jax.experimental.pallas import tpu as pltpu
from jax.experimental.pallas import tpu_sc as plsc

INPUT_DIM = 3
NUM_LEVELS = 16
LEVEL_DIM = 2
BASE_RES = 16
LOG2_HASHMAP_SIZE = 19
N_POINTS = 524288

P1 = np.int32(np.uint32(2654435761).astype(np.int64) - (1 << 32))
P2 = np.int32(805459861)
HASH_MASK = (1 << LOG2_HASHMAP_SIZE) - 1


def _level_table():
    max_params = 2 ** LOG2_HASHMAP_SIZE
    levels = []
    offset = 0
    for lvl in range(NUM_LEVELS):
        table_res = int(np.ceil(BASE_RES * 2.0 ** lvl)) + 1
        params = min(max_params, table_res ** INPUT_DIM)
        params = int(np.ceil(params / 8) * 8)
        scale = float(np.exp2(lvl) * BASE_RES - 1.0)
        res = int(np.ceil(scale)) + 1
        use_hash = (res ** INPUT_DIM) > params
        levels.append((scale, offset, params, use_hash, res))
        offset += params
    return levels, offset


LEVELS, TOTAL_ROWS = _level_table()

L01_ROWS = LEVELS[2][1]

NW = 32
PW = N_POINTS // NW
C = 1024
NG = C // 16
NCH = PW // C
OUT_DIM = NUM_LEVELS * LEVEL_DIM


def _corner_indices(ux, uy, uz, level):
    scale, off, hsize, use_hash, res = LEVELS[level]
    if use_hash:
        hx = (ux, ux + 1)
        hy0 = uy * P1
        hz0 = uz * P2
        hy = (hy0, hy0 + P1)
        hz = (hz0, hz0 + P2)
        hyz = [[hy[by] ^ hz[bz] for bz in range(2)] for by in range(2)]
        out = []
        for c in range(8):
            b0, b1, b2 = c & 1, (c >> 1) & 1, (c >> 2) & 1
            out.append(((hx[b0] ^ hyz[b1][b2]) & HASH_MASK) + off)
        return out
    e1 = np.int32(res)
    e2 = np.int32(res * res)
    cx = (ux, ux + 1)
    ty0 = uy * e1
    tz0 = uz * e2
    ty = (ty0, ty0 + e1)
    tz = (tz0, tz0 + e2)
    tyz = [[ty[by] + tz[bz] for bz in range(2)] for by in range(2)]
    out = []
    for c in range(8):
        b0, b1, b2 = c & 1, (c >> 1) & 1, (c >> 2) & 1
        idx = cx[b0] + tyz[b1][b2]
        idx = jnp.where(idx >= hsize, idx - hsize, idx)
        out.append(idx + off)
    return out


def _make_grid_kernel():
    mesh = plsc.VectorSubcoreMesh(core_axis_name="c", subcore_axis_name="s")

    @functools.partial(
        pl.kernel,
        mesh=mesh,
        compiler_params=pltpu.CompilerParams(needs_layout_passes=False),
        out_type=jax.ShapeDtypeStruct((N_POINTS * OUT_DIM,), jnp.float32),
        scratch_types=[
            pltpu.VMEM((C,), jnp.float32),
            pltpu.VMEM((C,), jnp.float32),
            pltpu.VMEM((C,), jnp.float32),
            pltpu.VMEM((8 * C,), jnp.int32),
            pltpu.VMEM((8 * C,), jnp.int32),
            pltpu.VMEM((8 * C,), jnp.float32),
            pltpu.VMEM((8 * C,), jnp.float32),
            pltpu.VMEM((8 * C,), jnp.int32),
            pltpu.VMEM((8 * C,), jnp.int32),
            pltpu.VMEM((C * OUT_DIM,), jnp.float32),
            pltpu.VMEM((L01_ROWS,), jnp.int32),
            pltpu.SemaphoreType.DMA,
            pltpu.SemaphoreType.DMA,
        ],
    )
    def grid_kernel(xs_h, ys_h, zs_h, tab_h, out_h,
                    xs_v, ys_v, zs_v, idx_a, idx_b, w_a, w_b,
                    rows_a, rows_b, ob_v, ltab_v, sem_a, sem_b):
        wid = lax.axis_index("c") * 16 + lax.axis_index("s")
        iota = lax.iota(jnp.int32, 16)
        iota8 = iota * 8
        orow = iota * OUT_DIM
        half = np.float32(0.5)
        one = np.float32(1.0)
        bufs = ((idx_a, w_a, rows_a, sem_a), (idx_b, w_b, rows_b, sem_b))

        def geom(g, level):
            a = np.float32(LEVELS[level][0])
            p0 = g * 16
            xv = xs_v[pl.ds(p0, 16)]
            yv = ys_v[pl.ds(p0, 16)]
            zv = zs_v[pl.ds(p0, 16)]
            px = ((xv + one) * half) * a + half
            py = ((yv + one) * half) * a + half
            pz = ((zv + one) * half) * a + half
            ux = px.astype(jnp.int32)
            uy = py.astype(jnp.int32)
            uz = pz.astype(jnp.int32)
            fx = px - ux.astype(jnp.float32)
            fy = py - uy.astype(jnp.float32)
            fz = pz - uz.astype(jnp.float32)
            idxs = _corner_indices(ux, uy, uz, level)
            gx = (1.0 - fx, fx)
            gy = (1.0 - fy, fy)
            gz = (1.0 - fz, fz)
            wxy = [[gx[b0] * gy[b1] for b1 in range(2)] for b0 in range(2)]
            ws = [wxy[c & 1][(c >> 1) & 1] * gz[(c >> 2) & 1]
                  for c in range(8)]
            return p0, idxs, ws

        def compute_pass(level, idx_v, w_v):
            def grp_body(g, carry2):
                p0, idxs, ws = geom(g, level)
                gbase = p0 * 8 + iota8
                for c in range(8):
                    plsc.store_scatter(idx_v, [gbase + c], idxs[c])
                    w_v[pl.ds(c * C + p0, 16)] = ws[c]
                return carry2

            lax.fori_loop(0, NG, grp_body, 0, unroll=False)

        def local_pass(level):
            obase = 2 * level

            def grp_body(g, carry2):
                p0, idxs, ws = geom(g, level)
                acc0 = jnp.zeros((16,), jnp.float32)
                acc1 = jnp.zeros((16,), jnp.float32)
                for c in range(8):
                    gv = plsc.load_gather(ltab_v, [idxs[c]])
                    f0, f1 = plsc.unpack(
                        plsc.bitcast(gv, jnp.bfloat16),
                        format=plsc.PackFormat.INTERLEAVED)
                    acc0 = acc0 + ws[c] * f0
                    acc1 = acc1 + ws[c] * f1
                opos = p0 * OUT_DIM + obase + orow
                plsc.store_scatter(ob_v, [opos], acc0)
                plsc.store_scatter(ob_v, [opos + 1], acc1)
                return carry2

            lax.fori_loop(0, NG, grp_body, 0, unroll=False)

        def acc_pass(level, rows_v, w_v):
            obase = 2 * level

            def acc_body(g, carry2):
                p0 = g * 16
                acc0 = jnp.zeros((16,), jnp.float32)
                acc1 = jnp.zeros((16,), jnp.float32)
                gbase = p0 * 8 + iota8
                for c in range(8):
                    rbase = c * C + p0
                    wv = w_v[pl.ds(rbase, 16)]
                    gv = plsc.load_gather(rows_v, [gbase + c])
                    f0, f1 = plsc.unpack(
                        plsc.bitcast(gv, jnp.bfloat16),
                        format=plsc.PackFormat.INTERLEAVED)
                    acc0 = acc0 + wv * f0
                    acc1 = acc1 + wv * f1
                opos = p0 * OUT_DIM + obase + orow
                plsc.store_scatter(ob_v, [opos], acc0)
                plsc.store_scatter(ob_v, [opos + 1], acc1)
                return carry2

            lax.fori_loop(0, NG, acc_body, 0, unroll=False)

        def chunk_body(ch, carry):
            base = wid * PW + ch * C
            pltpu.sync_copy(xs_h.at[pl.ds(base, C)], xs_v)
            pltpu.sync_copy(ys_h.at[pl.ds(base, C)], ys_v)
            pltpu.sync_copy(zs_h.at[pl.ds(base, C)], zs_v)

            compute_pass(2, bufs[0][0], bufs[0][1])
            cps = {2: pltpu.async_copy(tab_h.at[bufs[0][0]], bufs[0][2],
                                       bufs[0][3])}
            compute_pass(3, bufs[1][0], bufs[1][1])
            cps[3] = pltpu.async_copy(tab_h.at[bufs[1][0]], bufs[1][2],
                                      bufs[1][3])
            local_pass(0)
            local_pass(1)
            for level in range(2, NUM_LEVELS):
                cur = bufs[level % 2]
                cps.pop(level).wait()
                acc_pass(level, cur[2], cur[1])
                if level + 2 < NUM_LEVELS:
                    compute_pass(level + 2, cur[0], cur[1])
                    cps[level + 2] = pltpu.async_copy(
                        tab_h.at[cur[0]], cur[2], cur[3])

            pltpu.sync_copy(ob_v, out_h.at[pl.ds(base * OUT_DIM, C * OUT_DIM)])
            return carry

        pltpu.sync_copy(tab_h.at[pl.ds(0, L01_ROWS)], ltab_v)
        lax.fori_loop(0, NCH, chunk_body, 0, unroll=False)

    return grid_kernel


_GRID_KERNEL_CACHE = []


def kernel(inputs, embeddings):
    if not _GRID_KERNEL_CACHE:
        _GRID_KERNEL_CACHE.append(_make_grid_kernel())
    xyz = inputs.T
    packed = lax.bitcast_convert_type(
        embeddings.astype(jnp.bfloat16), jnp.int32)
    flat = _GRID_KERNEL_CACHE[0](xyz[0], xyz[1], xyz[2], packed)
    return flat.reshape(N_POINTS, OUT_DIM)
